# baseline (speedup 1.0000x reference)
import jax
import jax.numpy as jnp
from jax import lax
from jax.experimental import pallas as pl
from jax.experimental.pallas import tpu as pltpu
from jax.experimental.pallas import tpu_sc as plsc

N = 10000
E = 320000
D_IN = 128
D_HID = 200
D_PAD = 256

NC = 2
NS = 16
HALF = D_PAD // NC
CHUNK = 128
CPT = 160
E_PAD = NS * CPT * CHUNK
N_PAD = 10240
RPT = N_PAD // NS
ZR = 16
ROW_BLK = 1280


def _prep_edges(edge):
    src = edge[0].astype(jnp.int32)
    dst = edge[1].astype(jnp.int32)
    pad = E_PAD - E
    src = jnp.concatenate([src, jnp.zeros((pad,), jnp.int32)])
    dst = jnp.concatenate([dst, jnp.full((pad,), N, jnp.int32)])
    return src.reshape(NS, CPT, CHUNK), dst.reshape(NS, CPT, CHUNK)


def _sc_agg_body(hs_lo, hs_hi, hr_lo, hr_hi, ss, sd, rs, rd, agg_lo, agg_hi,
                 src_v, dst_v, rows0, rows1, zb_v, acc_sh,
                 gsem0, gsem1, ssem0, ssem1):
    rows = (rows0, rows1)
    gsem = (gsem0, gsem1)
    ssem = (ssem0, ssem1)
    cid = lax.axis_index("c")
    sid = lax.axis_index("s")
    base = sid * RPT

    zv = jnp.zeros((16,), jnp.float32)
    for i in range(ZR):
        for k in range(HALF // 16):
            zb_v[i, pl.ds(k * 16, 16)] = zv

    def zcopy(i, c):
        pltpu.sync_copy(zb_v, acc_sh.at[pl.ds(base + i * ZR, ZR)])
        return c
    lax.fori_loop(0, RPT // ZR, zcopy, 0)

    plsc.subcore_barrier()

    def run(h_s, h_r, agg):
        def one_set(src_hbm, dst_hbm, h_hbm):
            def body(j, c):
                pltpu.sync_copy(src_hbm.at[sid, j], src_v)
                pltpu.sync_copy(dst_hbm.at[sid, j], dst_v)
                pltpu.async_copy(h_hbm.at[src_v], rows[0], gsem[0]).wait()
                pltpu.sync_copy(rows[0], acc_sh.at[dst_v], add=True)
                return c
            lax.fori_loop(0, CPT, body, 0)

        one_set(ss, sd, h_s)
        one_set(rs, rd, h_r)
        plsc.subcore_barrier()
        pltpu.sync_copy(acc_sh.at[pl.ds(base, RPT)], agg.at[pl.ds(base, RPT)])

    @pl.when(cid == 0)
    def _():
        run(hs_lo, hr_lo, agg_lo)

    @pl.when(cid == 1)
    def _():
        run(hs_hi, hr_hi, agg_hi)


_half = jax.ShapeDtypeStruct((N_PAD, HALF), jnp.float32)

_sc_agg = pl.kernel(
    _sc_agg_body,
    out_type=[_half, _half],
    mesh=plsc.VectorSubcoreMesh(core_axis_name="c", subcore_axis_name="s"),
    scratch_types=[
        pltpu.VMEM((CHUNK,), jnp.int32),
        pltpu.VMEM((CHUNK,), jnp.int32),
        pltpu.VMEM((CHUNK, HALF), jnp.float32),
        pltpu.VMEM((CHUNK, HALF), jnp.float32),
        pltpu.VMEM((ZR, HALF), jnp.float32),
        pltpu.VMEM_SHARED((N_PAD, HALF), jnp.float32),
        pltpu.SemaphoreType.DMA,
        pltpu.SemaphoreType.DMA,
        pltpu.SemaphoreType.DMA,
        pltpu.SemaphoreType.DMA,
    ],
)


def _mm3_kernel(x_ref, ws_ref, wr_ref, wf_ref, bs_ref, br_ref, bf_ref,
                osl_ref, osh_ref, orl_ref, orh_ref, of_ref):
    xb = x_ref[...]
    ts = jnp.dot(xb, ws_ref[...], preferred_element_type=jnp.float32) + bs_ref[...]
    tr = jnp.dot(xb, wr_ref[...], preferred_element_type=jnp.float32) + br_ref[...]
    osl_ref[...] = ts[:, :HALF]
    osh_ref[...] = ts[:, HALF:]
    orl_ref[...] = tr[:, :HALF]
    orh_ref[...] = tr[:, HALF:]
    of_ref[...] = jnp.dot(xb, wf_ref[...], preferred_element_type=jnp.float32) + bf_ref[...]


def _relu_mm3_kernel(f_ref, al_ref, ah_ref, ws_ref, wr_ref, wf_ref,
                     bs_ref, br_ref, bf_ref,
                     osl_ref, osh_ref, orl_ref, orh_ref, of_ref):
    agg = jnp.concatenate([al_ref[...], ah_ref[...]], axis=1)
    h = jnp.maximum(f_ref[...] + agg, 0.0)
    ts = jnp.dot(h, ws_ref[...], preferred_element_type=jnp.float32) + bs_ref[...]
    tr = jnp.dot(h, wr_ref[...], preferred_element_type=jnp.float32) + br_ref[...]
    osl_ref[...] = ts[:, :HALF]
    osh_ref[...] = ts[:, HALF:]
    orl_ref[...] = tr[:, :HALF]
    orh_ref[...] = tr[:, HALF:]
    of_ref[...] = jnp.dot(h, wf_ref[...], preferred_element_type=jnp.float32) + bf_ref[...]


def _combine_kernel(f_ref, al_ref, ah_ref, o_ref):
    o_ref[...] = f_ref[...] + jnp.concatenate([al_ref[...], ah_ref[...]],
                                              axis=1)


def _blk(w):
    return pl.BlockSpec((ROW_BLK, w), lambda i: (i, 0))


def _mm3(lead, lead_w, Ws, Wr, Wf, bs, br, bf, K, relu_combine=False):
    grid = (N_PAD // ROW_BLK,)
    lead_specs = [_blk(w) for w in lead_w]
    w_specs = [pl.BlockSpec((K, D_PAD), lambda i: (0, 0))] * 3
    b_specs = [pl.BlockSpec((1, D_PAD), lambda i: (0, 0))] * 3
    body = _relu_mm3_kernel if relu_combine else _mm3_kernel
    return pl.pallas_call(
        body,
        grid=grid,
        in_specs=lead_specs + w_specs + b_specs,
        out_specs=[_blk(HALF)] * 4 + [_blk(D_PAD)],
        out_shape=[_half] * 4 + [jax.ShapeDtypeStruct((N_PAD, D_PAD),
                                                      jnp.float32)],
    )(*lead, Ws, Wr, Wf, bs, br, bf)


def _combine(f, a_lo, a_hi):
    grid = (N_PAD // ROW_BLK,)
    return pl.pallas_call(
        _combine_kernel,
        grid=grid,
        in_specs=[_blk(D_PAD), _blk(HALF), _blk(HALF)],
        out_specs=_blk(D_PAD),
        out_shape=jax.ShapeDtypeStruct((N_PAD, D_PAD), jnp.float32),
    )(f, a_lo, a_hi)


def _pad_w(W, K_PAD):
    return jnp.pad(W, ((0, K_PAD - W.shape[0]), (0, D_PAD - W.shape[1])))


def _pad_b(b):
    return jnp.pad(b, (0, D_PAD - b.shape[0])).reshape(1, D_PAD)


def kernel(edge_sim, edge_rat, x,
           W1s, b1s, W1r, b1r, W1f, b1f,
           W2s, b2s, W2r, b2r, W2f, b2f):
    ss, sd = _prep_edges(edge_sim)
    rs, rd = _prep_edges(edge_rat)
    x_p = jnp.pad(x, ((0, N_PAD - N), (0, 0)))

    hsl, hsh, hrl, hrh, f1 = _mm3(
        (x_p,), (D_IN,), _pad_w(W1s, D_IN), _pad_w(W1r, D_IN),
        _pad_w(W1f, D_IN), _pad_b(b1s), _pad_b(b1r), _pad_b(b1f), D_IN)
    a1l, a1h = _sc_agg(hsl, hsh, hrl, hrh, ss, sd, rs, rd)
    hsl2, hsh2, hrl2, hrh2, f2 = _mm3(
        (f1, a1l, a1h), (D_PAD, HALF, HALF), _pad_w(W2s, D_PAD),
        _pad_w(W2r, D_PAD), _pad_w(W2f, D_PAD), _pad_b(b2s), _pad_b(b2r),
        _pad_b(b2f), D_PAD, relu_combine=True)
    a2l, a2h = _sc_agg(hsl2, hsh2, hrl2, hrh2, ss, sd, rs, rd)
    out = _combine(f2, a2l, a2h)
    return out[:N, :D_HID]


# speedup vs baseline: 1.2770x; 1.2770x over previous
import jax
import jax.numpy as jnp
from jax import lax
from jax.experimental import pallas as pl
from jax.experimental.pallas import tpu as pltpu
from jax.experimental.pallas import tpu_sc as plsc

N = 10000
E = 320000
D_IN = 128
D_HID = 200
D_PAD = 256

NC = 2
NS = 16
HALF = D_PAD // NC
CHUNK = 128
CPT = 160
IB = 16
NIB = CPT // IB
E_PAD = NS * CPT * CHUNK
N_PAD = 10240
RPT = N_PAD // NS
ZR = 16
ROW_BLK = 1280


def _prep_edges(edge):
    src = edge[0].astype(jnp.int32)
    dst = edge[1].astype(jnp.int32)
    pad = E_PAD - E
    src = jnp.concatenate([src, jnp.zeros((pad,), jnp.int32)])
    dst = jnp.concatenate([dst, jnp.full((pad,), N, jnp.int32)])
    return src.reshape(NS, CPT, CHUNK), dst.reshape(NS, CPT, CHUNK)


def _sc_agg_body(hs_lo, hs_hi, hr_lo, hr_hi, ss, sd, rs, rd, agg_lo, agg_hi,
                 src_v, dst_v, rows0, rows1, zb_v, acc_sh,
                 gsem0, gsem1, ssem0, ssem1):
    rows = (rows0, rows1)
    gsem = (gsem0, gsem1)
    ssem = (ssem0, ssem1)
    cid = lax.axis_index("c")
    sid = lax.axis_index("s")
    base = sid * RPT

    zv = jnp.zeros((16,), jnp.float32)
    for i in range(ZR):
        for k in range(HALF // 16):
            zb_v[i, pl.ds(k * 16, 16)] = zv

    def zcopy(i, c):
        pltpu.sync_copy(zb_v, acc_sh.at[pl.ds(base + i * ZR, ZR)])
        return c
    lax.fori_loop(0, RPT // ZR, zcopy, 0)

    plsc.subcore_barrier()

    def run(h_s, h_r, agg):
        def one_set(src_hbm, dst_hbm, h_hbm):
            def wait_g(bi):
                pltpu.make_async_copy(h_hbm.at[pl.ds(0, CHUNK)], rows[bi],
                                      gsem[bi]).wait()

            def wait_s(bi):
                pltpu.make_async_copy(rows[bi], acc_sh.at[pl.ds(0, CHUNK)],
                                      ssem[bi]).wait()

            def fire_g(j, bi):
                pltpu.async_copy(h_hbm.at[src_v.at[j]], rows[bi], gsem[bi])

            def fire_s(j, bi):
                pltpu.async_copy(rows[bi], acc_sh.at[dst_v.at[j]], ssem[bi],
                                 add=True)

            def blk(b, c):
                pltpu.sync_copy(src_hbm.at[sid, pl.ds(b * IB, IB)], src_v)
                pltpu.sync_copy(dst_hbm.at[sid, pl.ds(b * IB, IB)], dst_v)
                fire_g(0, 0)

                def pair(p, c2):
                    j0 = 2 * p
                    wait_g(0)

                    @pl.when(p > 0)
                    def _():
                        wait_s(1)
                    fire_g(j0 + 1, 1)
                    fire_s(j0, 0)
                    wait_g(1)
                    wait_s(0)

                    @pl.when(p < IB // 2 - 1)
                    def _():
                        fire_g(j0 + 2, 0)
                    fire_s(j0 + 1, 1)
                    return c2
                lax.fori_loop(0, IB // 2, pair, c)
                wait_s(1)
                return c
            lax.fori_loop(0, NIB, blk, 0)

        one_set(ss, sd, h_s)
        one_set(rs, rd, h_r)
        plsc.subcore_barrier()
        pltpu.sync_copy(acc_sh.at[pl.ds(base, RPT)], agg.at[pl.ds(base, RPT)])

    @pl.when(cid == 0)
    def _():
        run(hs_lo, hr_lo, agg_lo)

    @pl.when(cid == 1)
    def _():
        run(hs_hi, hr_hi, agg_hi)


_half = jax.ShapeDtypeStruct((N_PAD, HALF), jnp.float32)

_sc_agg = pl.kernel(
    _sc_agg_body,
    out_type=[_half, _half],
    mesh=plsc.VectorSubcoreMesh(core_axis_name="c", subcore_axis_name="s"),
    scratch_types=[
        pltpu.VMEM((IB, CHUNK), jnp.int32),
        pltpu.VMEM((IB, CHUNK), jnp.int32),
        pltpu.VMEM((CHUNK, HALF), jnp.float32),
        pltpu.VMEM((CHUNK, HALF), jnp.float32),
        pltpu.VMEM((ZR, HALF), jnp.float32),
        pltpu.VMEM_SHARED((N_PAD, HALF), jnp.float32),
        pltpu.SemaphoreType.DMA,
        pltpu.SemaphoreType.DMA,
        pltpu.SemaphoreType.DMA,
        pltpu.SemaphoreType.DMA,
    ],
)


def _mm3_kernel(x_ref, ws_ref, wr_ref, wf_ref, bs_ref, br_ref, bf_ref,
                osl_ref, osh_ref, orl_ref, orh_ref, of_ref):
    xb = x_ref[...]
    ts = jnp.dot(xb, ws_ref[...], preferred_element_type=jnp.float32) + bs_ref[...]
    tr = jnp.dot(xb, wr_ref[...], preferred_element_type=jnp.float32) + br_ref[...]
    osl_ref[...] = ts[:, :HALF]
    osh_ref[...] = ts[:, HALF:]
    orl_ref[...] = tr[:, :HALF]
    orh_ref[...] = tr[:, HALF:]
    of_ref[...] = jnp.dot(xb, wf_ref[...], preferred_element_type=jnp.float32) + bf_ref[...]


def _relu_mm3_kernel(f_ref, al_ref, ah_ref, ws_ref, wr_ref, wf_ref,
                     bs_ref, br_ref, bf_ref,
                     osl_ref, osh_ref, orl_ref, orh_ref, of_ref):
    agg = jnp.concatenate([al_ref[...], ah_ref[...]], axis=1)
    h = jnp.maximum(f_ref[...] + agg, 0.0)
    ts = jnp.dot(h, ws_ref[...], preferred_element_type=jnp.float32) + bs_ref[...]
    tr = jnp.dot(h, wr_ref[...], preferred_element_type=jnp.float32) + br_ref[...]
    osl_ref[...] = ts[:, :HALF]
    osh_ref[...] = ts[:, HALF:]
    orl_ref[...] = tr[:, :HALF]
    orh_ref[...] = tr[:, HALF:]
    of_ref[...] = jnp.dot(h, wf_ref[...], preferred_element_type=jnp.float32) + bf_ref[...]


def _combine_kernel(f_ref, al_ref, ah_ref, o_ref):
    o_ref[...] = f_ref[...] + jnp.concatenate([al_ref[...], ah_ref[...]],
                                              axis=1)


def _blk(w):
    return pl.BlockSpec((ROW_BLK, w), lambda i: (i, 0))


def _mm3(lead, lead_w, Ws, Wr, Wf, bs, br, bf, K, relu_combine=False):
    grid = (N_PAD // ROW_BLK,)
    lead_specs = [_blk(w) for w in lead_w]
    w_specs = [pl.BlockSpec((K, D_PAD), lambda i: (0, 0))] * 3
    b_specs = [pl.BlockSpec((1, D_PAD), lambda i: (0, 0))] * 3
    body = _relu_mm3_kernel if relu_combine else _mm3_kernel
    return pl.pallas_call(
        body,
        grid=grid,
        in_specs=lead_specs + w_specs + b_specs,
        out_specs=[_blk(HALF)] * 4 + [_blk(D_PAD)],
        out_shape=[_half] * 4 + [jax.ShapeDtypeStruct((N_PAD, D_PAD),
                                                      jnp.float32)],
    )(*lead, Ws, Wr, Wf, bs, br, bf)


def _combine(f, a_lo, a_hi):
    grid = (N_PAD // ROW_BLK,)
    return pl.pallas_call(
        _combine_kernel,
        grid=grid,
        in_specs=[_blk(D_PAD), _blk(HALF), _blk(HALF)],
        out_specs=_blk(D_PAD),
        out_shape=jax.ShapeDtypeStruct((N_PAD, D_PAD), jnp.float32),
    )(f, a_lo, a_hi)


def _pad_w(W, K_PAD):
    return jnp.pad(W, ((0, K_PAD - W.shape[0]), (0, D_PAD - W.shape[1])))


def _pad_b(b):
    return jnp.pad(b, (0, D_PAD - b.shape[0])).reshape(1, D_PAD)


def kernel(edge_sim, edge_rat, x,
           W1s, b1s, W1r, b1r, W1f, b1f,
           W2s, b2s, W2r, b2r, W2f, b2f):
    ss, sd = _prep_edges(edge_sim)
    rs, rd = _prep_edges(edge_rat)
    x_p = jnp.pad(x, ((0, N_PAD - N), (0, 0)))

    hsl, hsh, hrl, hrh, f1 = _mm3(
        (x_p,), (D_IN,), _pad_w(W1s, D_IN), _pad_w(W1r, D_IN),
        _pad_w(W1f, D_IN), _pad_b(b1s), _pad_b(b1r), _pad_b(b1f), D_IN)
    a1l, a1h = _sc_agg(hsl, hsh, hrl, hrh, ss, sd, rs, rd)
    hsl2, hsh2, hrl2, hrh2, f2 = _mm3(
        (f1, a1l, a1h), (D_PAD, HALF, HALF), _pad_w(W2s, D_PAD),
        _pad_w(W2r, D_PAD), _pad_w(W2f, D_PAD), _pad_b(b2s), _pad_b(b2r),
        _pad_b(b2f), D_PAD, relu_combine=True)
    a2l, a2h = _sc_agg(hsl2, hsh2, hrl2, hrh2, ss, sd, rs, rd)
    out = _combine(f2, a2l, a2h)
    return out[:N, :D_HID]


# speedup vs baseline: 2.3838x; 1.8668x over previous
import jax
import jax.numpy as jnp
from jax import lax
from jax.experimental import pallas as pl
from jax.experimental.pallas import tpu as pltpu
from jax.experimental.pallas import tpu_sc as plsc

N = 10000
E = 320000
D_IN = 128
D_HID = 200
D_PAD = 256

NC = 2
NS = 16
HALF = D_PAD // NC
QW = 64
CHUNK = 128
CPT = 160
IB = 16
NIB = CPT // IB
E_PAD = NS * CPT * CHUNK
N_PAD = 10240
RPT = N_PAD // NS
ZR = 16
ROW_BLK = 1280


def _prep_edges(edge):
    src = edge[0].astype(jnp.int32)
    dst = edge[1].astype(jnp.int32)
    pad = E_PAD - E
    src = jnp.concatenate([src, jnp.zeros((pad,), jnp.int32)])
    dst = jnp.concatenate([dst, jnp.full((pad,), N, jnp.int32)])
    return src.reshape(NS, CPT, CHUNK), dst.reshape(NS, CPT, CHUNK)


def _sc_agg_body(hs_lo, hs_hi, hr_lo, hr_hi, ss, sd, rs, rd, agg_lo, agg_hi,
                 src_v, dst_v, rows0, rows1, zb_v, acc_sh, tbl_sh,
                 gsem0, gsem1, ssem0, ssem1):
    rows = (rows0, rows1)
    gsem = (gsem0, gsem1)
    ssem = (ssem0, ssem1)
    cid = lax.axis_index("c")
    sid = lax.axis_index("s")
    base = sid * RPT

    zv = jnp.zeros((16,), jnp.float32)
    for i in range(ZR):
        for k in range(QW // 16):
            zb_v[i, pl.ds(k * 16, 16)] = zv

    def one_set(src_hbm, dst_hbm):
        def wait_g(bi):
            pltpu.make_async_copy(tbl_sh.at[pl.ds(0, CHUNK)], rows[bi],
                                  gsem[bi]).wait()

        def wait_s(bi):
            pltpu.make_async_copy(rows[bi], acc_sh.at[pl.ds(0, CHUNK)],
                                  ssem[bi]).wait()

        def fire_g(j, bi):
            pltpu.async_copy(tbl_sh.at[src_v.at[j]], rows[bi], gsem[bi])

        def fire_s(j, bi):
            pltpu.async_copy(rows[bi], acc_sh.at[dst_v.at[j]], ssem[bi],
                             add=True)

        def blk(b, c):
            pltpu.sync_copy(src_hbm.at[sid, pl.ds(b * IB, IB)], src_v)
            pltpu.sync_copy(dst_hbm.at[sid, pl.ds(b * IB, IB)], dst_v)
            fire_g(0, 0)

            def pair(p, c2):
                j0 = 2 * p
                wait_g(0)

                @pl.when(p > 0)
                def _():
                    wait_s(1)
                fire_g(j0 + 1, 1)
                fire_s(j0, 0)
                wait_g(1)
                wait_s(0)

                @pl.when(p < IB // 2 - 1)
                def _():
                    fire_g(j0 + 2, 0)
                fire_s(j0 + 1, 1)
                return c2
            lax.fori_loop(0, IB // 2, pair, c)
            wait_s(1)
            return c
        lax.fori_loop(0, NIB, blk, 0)

    def run(h_s, h_r, agg):
        for q in range(HALF // QW):
            qc = q * QW

            def zcopy(i, c):
                pltpu.sync_copy(zb_v, acc_sh.at[pl.ds(base + i * ZR, ZR)])
                return c
            lax.fori_loop(0, RPT // ZR, zcopy, 0)
            pltpu.sync_copy(h_s.at[pl.ds(base, RPT), pl.ds(qc, QW)],
                            tbl_sh.at[pl.ds(base, RPT)])
            plsc.subcore_barrier()
            one_set(ss, sd)
            plsc.subcore_barrier()
            pltpu.sync_copy(h_r.at[pl.ds(base, RPT), pl.ds(qc, QW)],
                            tbl_sh.at[pl.ds(base, RPT)])
            plsc.subcore_barrier()
            one_set(rs, rd)
            plsc.subcore_barrier()
            pltpu.sync_copy(acc_sh.at[pl.ds(base, RPT)],
                            agg.at[pl.ds(base, RPT), pl.ds(qc, QW)])
            plsc.subcore_barrier()

    @pl.when(cid == 0)
    def _():
        run(hs_lo, hr_lo, agg_lo)

    @pl.when(cid == 1)
    def _():
        run(hs_hi, hr_hi, agg_hi)


_half = jax.ShapeDtypeStruct((N_PAD, HALF), jnp.float32)

_sc_agg = pl.kernel(
    _sc_agg_body,
    out_type=[_half, _half],
    mesh=plsc.VectorSubcoreMesh(core_axis_name="c", subcore_axis_name="s"),
    scratch_types=[
        pltpu.VMEM((IB, CHUNK), jnp.int32),
        pltpu.VMEM((IB, CHUNK), jnp.int32),
        pltpu.VMEM((CHUNK, QW), jnp.float32),
        pltpu.VMEM((CHUNK, QW), jnp.float32),
        pltpu.VMEM((ZR, QW), jnp.float32),
        pltpu.VMEM_SHARED((N_PAD, QW), jnp.float32),
        pltpu.VMEM_SHARED((N_PAD, QW), jnp.float32),
        pltpu.SemaphoreType.DMA,
        pltpu.SemaphoreType.DMA,
        pltpu.SemaphoreType.DMA,
        pltpu.SemaphoreType.DMA,
    ],
    compiler_params=pltpu.CompilerParams(use_tc_tiling_on_sc=False),
)


def _mm3_kernel(x_ref, ws_ref, wr_ref, wf_ref, bs_ref, br_ref, bf_ref,
                osl_ref, osh_ref, orl_ref, orh_ref, of_ref):
    xb = x_ref[...]
    ts = jnp.dot(xb, ws_ref[...], preferred_element_type=jnp.float32) + bs_ref[...]
    tr = jnp.dot(xb, wr_ref[...], preferred_element_type=jnp.float32) + br_ref[...]
    osl_ref[...] = ts[:, :HALF]
    osh_ref[...] = ts[:, HALF:]
    orl_ref[...] = tr[:, :HALF]
    orh_ref[...] = tr[:, HALF:]
    of_ref[...] = jnp.dot(xb, wf_ref[...], preferred_element_type=jnp.float32) + bf_ref[...]


def _relu_mm3_kernel(f_ref, al_ref, ah_ref, ws_ref, wr_ref, wf_ref,
                     bs_ref, br_ref, bf_ref,
                     osl_ref, osh_ref, orl_ref, orh_ref, of_ref):
    agg = jnp.concatenate([al_ref[...], ah_ref[...]], axis=1)
    h = jnp.maximum(f_ref[...] + agg, 0.0)
    ts = jnp.dot(h, ws_ref[...], preferred_element_type=jnp.float32) + bs_ref[...]
    tr = jnp.dot(h, wr_ref[...], preferred_element_type=jnp.float32) + br_ref[...]
    osl_ref[...] = ts[:, :HALF]
    osh_ref[...] = ts[:, HALF:]
    orl_ref[...] = tr[:, :HALF]
    orh_ref[...] = tr[:, HALF:]
    of_ref[...] = jnp.dot(h, wf_ref[...], preferred_element_type=jnp.float32) + bf_ref[...]


def _combine_kernel(f_ref, al_ref, ah_ref, o_ref):
    o_ref[...] = f_ref[...] + jnp.concatenate([al_ref[...], ah_ref[...]],
                                              axis=1)


def _blk(w):
    return pl.BlockSpec((ROW_BLK, w), lambda i: (i, 0))


def _mm3(lead, lead_w, Ws, Wr, Wf, bs, br, bf, K, relu_combine=False):
    grid = (N_PAD // ROW_BLK,)
    lead_specs = [_blk(w) for w in lead_w]
    w_specs = [pl.BlockSpec((K, D_PAD), lambda i: (0, 0))] * 3
    b_specs = [pl.BlockSpec((1, D_PAD), lambda i: (0, 0))] * 3
    body = _relu_mm3_kernel if relu_combine else _mm3_kernel
    return pl.pallas_call(
        body,
        grid=grid,
        in_specs=lead_specs + w_specs + b_specs,
        out_specs=[_blk(HALF)] * 4 + [_blk(D_PAD)],
        out_shape=[_half] * 4 + [jax.ShapeDtypeStruct((N_PAD, D_PAD),
                                                      jnp.float32)],
    )(*lead, Ws, Wr, Wf, bs, br, bf)


def _combine(f, a_lo, a_hi):
    grid = (N_PAD // ROW_BLK,)
    return pl.pallas_call(
        _combine_kernel,
        grid=grid,
        in_specs=[_blk(D_PAD), _blk(HALF), _blk(HALF)],
        out_specs=_blk(D_PAD),
        out_shape=jax.ShapeDtypeStruct((N_PAD, D_PAD), jnp.float32),
    )(f, a_lo, a_hi)


def _pad_w(W, K_PAD):
    return jnp.pad(W, ((0, K_PAD - W.shape[0]), (0, D_PAD - W.shape[1])))


def _pad_b(b):
    return jnp.pad(b, (0, D_PAD - b.shape[0])).reshape(1, D_PAD)


def kernel(edge_sim, edge_rat, x,
           W1s, b1s, W1r, b1r, W1f, b1f,
           W2s, b2s, W2r, b2r, W2f, b2f):
    ss, sd = _prep_edges(edge_sim)
    rs, rd = _prep_edges(edge_rat)
    x_p = jnp.pad(x, ((0, N_PAD - N), (0, 0)))

    hsl, hsh, hrl, hrh, f1 = _mm3(
        (x_p,), (D_IN,), _pad_w(W1s, D_IN), _pad_w(W1r, D_IN),
        _pad_w(W1f, D_IN), _pad_b(b1s), _pad_b(b1r), _pad_b(b1f), D_IN)
    a1l, a1h = _sc_agg(hsl, hsh, hrl, hrh, ss, sd, rs, rd)
    hsl2, hsh2, hrl2, hrh2, f2 = _mm3(
        (f1, a1l, a1h), (D_PAD, HALF, HALF), _pad_w(W2s, D_PAD),
        _pad_w(W2r, D_PAD), _pad_w(W2f, D_PAD), _pad_b(b2s), _pad_b(b2r),
        _pad_b(b2f), D_PAD, relu_combine=True)
    a2l, a2h = _sc_agg(hsl2, hsh2, hrl2, hrh2, ss, sd, rs, rd)
    out = _combine(f2, a2l, a2h)
    return out[:N, :D_HID]


# speedup vs baseline: 3.8431x; 1.6122x over previous
import jax
import jax.numpy as jnp
from jax import lax
from jax.experimental import pallas as pl
from jax.experimental.pallas import tpu as pltpu
from jax.experimental.pallas import tpu_sc as plsc

N = 10000
E = 320000
D_IN = 128
D_HID = 200
D_PAD = 256

NC = 2
NS = 16
HALF = D_PAD // NC
QW = 64
CHUNK = 128
CPT = 160
IB = 16
NIB = CPT // IB
E_PAD = NS * CPT * CHUNK
N_PAD = 10240
RPT = N_PAD // NS
ZR = 16
ROW_BLK = 1280


def _prep_edges(edge):
    src = edge[0].astype(jnp.int32)
    dst = edge[1].astype(jnp.int32)
    pad = E_PAD - E
    src = jnp.concatenate([src, jnp.zeros((pad,), jnp.int32)])
    dst = jnp.concatenate([dst, jnp.full((pad,), N, jnp.int32)])
    return src.reshape(NS, CPT, CHUNK), dst.reshape(NS, CPT, CHUNK)


def _sc_agg_body(hs_lo, hs_hi, hr_lo, hr_hi, ss, sd, rs, rd, agg_lo, agg_hi,
                 src_v, dst_v, rows0, rows1, zb_v, acc_sh, tbl_sh,
                 gsem0, gsem1, ssem0, ssem1):
    rows = (rows0, rows1)
    gsem = (gsem0, gsem1)
    ssem = (ssem0, ssem1)
    cid = lax.axis_index("c")
    sid = lax.axis_index("s")
    base = sid * RPT

    zv = jnp.zeros((32,), jnp.bfloat16)
    for i in range(ZR):
        for k in range(HALF // 32):
            zb_v[i, pl.ds(k * 32, 32)] = zv

    def one_set(src_hbm, dst_hbm):
        def wait_g(bi):
            pltpu.make_async_copy(tbl_sh.at[pl.ds(0, CHUNK)], rows[bi],
                                  gsem[bi]).wait()

        def wait_s(bi):
            pltpu.make_async_copy(rows[bi], acc_sh.at[pl.ds(0, CHUNK)],
                                  ssem[bi]).wait()

        def fire_g(j, bi):
            pltpu.async_copy(tbl_sh.at[src_v.at[j]], rows[bi], gsem[bi])

        def fire_s(j, bi):
            pltpu.async_copy(rows[bi], acc_sh.at[dst_v.at[j]], ssem[bi],
                             add=True)

        def blk(b, c):
            pltpu.sync_copy(src_hbm.at[sid, pl.ds(b * IB, IB)], src_v)
            pltpu.sync_copy(dst_hbm.at[sid, pl.ds(b * IB, IB)], dst_v)
            fire_g(0, 0)

            def pair(p, c2):
                j0 = 2 * p
                wait_g(0)

                @pl.when(p > 0)
                def _():
                    wait_s(1)
                fire_g(j0 + 1, 1)
                fire_s(j0, 0)
                wait_g(1)
                wait_s(0)

                @pl.when(p < IB // 2 - 1)
                def _():
                    fire_g(j0 + 2, 0)
                fire_s(j0 + 1, 1)
                return c2
            lax.fori_loop(0, IB // 2, pair, c)
            wait_s(1)
            return c
        lax.fori_loop(0, NIB, blk, 0)

    def run(h_s, h_r, agg):
        def zcopy(i, c):
            pltpu.sync_copy(zb_v, acc_sh.at[pl.ds(base + i * ZR, ZR)])
            return c
        lax.fori_loop(0, RPT // ZR, zcopy, 0)
        pltpu.sync_copy(h_s.at[pl.ds(base, RPT)], tbl_sh.at[pl.ds(base, RPT)])
        plsc.subcore_barrier()
        one_set(ss, sd)
        plsc.subcore_barrier()
        pltpu.sync_copy(h_r.at[pl.ds(base, RPT)], tbl_sh.at[pl.ds(base, RPT)])
        plsc.subcore_barrier()
        one_set(rs, rd)
        plsc.subcore_barrier()
        pltpu.sync_copy(acc_sh.at[pl.ds(base, RPT)], agg.at[pl.ds(base, RPT)])

    @pl.when(cid == 0)
    def _():
        run(hs_lo, hr_lo, agg_lo)

    @pl.when(cid == 1)
    def _():
        run(hs_hi, hr_hi, agg_hi)


_half = jax.ShapeDtypeStruct((N_PAD, HALF), jnp.bfloat16)

_sc_agg = pl.kernel(
    _sc_agg_body,
    out_type=[_half, _half],
    mesh=plsc.VectorSubcoreMesh(core_axis_name="c", subcore_axis_name="s"),
    scratch_types=[
        pltpu.VMEM((IB, CHUNK), jnp.int32),
        pltpu.VMEM((IB, CHUNK), jnp.int32),
        pltpu.VMEM((CHUNK, HALF), jnp.bfloat16),
        pltpu.VMEM((CHUNK, HALF), jnp.bfloat16),
        pltpu.VMEM((ZR, HALF), jnp.bfloat16),
        pltpu.VMEM_SHARED((N_PAD, HALF), jnp.bfloat16),
        pltpu.VMEM_SHARED((N_PAD, HALF), jnp.bfloat16),
        pltpu.SemaphoreType.DMA,
        pltpu.SemaphoreType.DMA,
        pltpu.SemaphoreType.DMA,
        pltpu.SemaphoreType.DMA,
    ],
    compiler_params=pltpu.CompilerParams(use_tc_tiling_on_sc=False),
)


def _mm3_kernel(x_ref, ws_ref, wr_ref, wf_ref, bs_ref, br_ref, bf_ref,
                osl_ref, osh_ref, orl_ref, orh_ref, of_ref):
    xb = x_ref[...]
    ts = jnp.dot(xb, ws_ref[...], preferred_element_type=jnp.float32) + bs_ref[...]
    tr = jnp.dot(xb, wr_ref[...], preferred_element_type=jnp.float32) + br_ref[...]
    osl_ref[...] = ts[:, :HALF].astype(jnp.bfloat16)
    osh_ref[...] = ts[:, HALF:].astype(jnp.bfloat16)
    orl_ref[...] = tr[:, :HALF].astype(jnp.bfloat16)
    orh_ref[...] = tr[:, HALF:].astype(jnp.bfloat16)
    of_ref[...] = jnp.dot(xb, wf_ref[...], preferred_element_type=jnp.float32) + bf_ref[...]


def _relu_mm3_kernel(f_ref, al_ref, ah_ref, ws_ref, wr_ref, wf_ref,
                     bs_ref, br_ref, bf_ref,
                     osl_ref, osh_ref, orl_ref, orh_ref, of_ref):
    agg = jnp.concatenate([al_ref[...], ah_ref[...]],
                          axis=1).astype(jnp.float32)
    h = jnp.maximum(f_ref[...] + agg, 0.0)
    ts = jnp.dot(h, ws_ref[...], preferred_element_type=jnp.float32) + bs_ref[...]
    tr = jnp.dot(h, wr_ref[...], preferred_element_type=jnp.float32) + br_ref[...]
    osl_ref[...] = ts[:, :HALF].astype(jnp.bfloat16)
    osh_ref[...] = ts[:, HALF:].astype(jnp.bfloat16)
    orl_ref[...] = tr[:, :HALF].astype(jnp.bfloat16)
    orh_ref[...] = tr[:, HALF:].astype(jnp.bfloat16)
    of_ref[...] = jnp.dot(h, wf_ref[...], preferred_element_type=jnp.float32) + bf_ref[...]


def _combine_kernel(f_ref, al_ref, ah_ref, o_ref):
    agg = jnp.concatenate([al_ref[...], ah_ref[...]],
                          axis=1).astype(jnp.float32)
    o_ref[...] = f_ref[...] + agg


def _blk(w):
    return pl.BlockSpec((ROW_BLK, w), lambda i: (i, 0))


def _mm3(lead, lead_w, Ws, Wr, Wf, bs, br, bf, K, relu_combine=False):
    grid = (N_PAD // ROW_BLK,)
    lead_specs = [_blk(w) for w in lead_w]
    w_specs = [pl.BlockSpec((K, D_PAD), lambda i: (0, 0))] * 3
    b_specs = [pl.BlockSpec((1, D_PAD), lambda i: (0, 0))] * 3
    body = _relu_mm3_kernel if relu_combine else _mm3_kernel
    return pl.pallas_call(
        body,
        grid=grid,
        in_specs=lead_specs + w_specs + b_specs,
        out_specs=[_blk(HALF)] * 4 + [_blk(D_PAD)],
        out_shape=[_half] * 4 + [jax.ShapeDtypeStruct((N_PAD, D_PAD),
                                                      jnp.float32)],
    )(*lead, Ws, Wr, Wf, bs, br, bf)


def _combine(f, a_lo, a_hi):
    grid = (N_PAD // ROW_BLK,)
    return pl.pallas_call(
        _combine_kernel,
        grid=grid,
        in_specs=[_blk(D_PAD), _blk(HALF), _blk(HALF)],
        out_specs=_blk(D_PAD),
        out_shape=jax.ShapeDtypeStruct((N_PAD, D_PAD), jnp.float32),
    )(f, a_lo, a_hi)


def _pad_w(W, K_PAD):
    return jnp.pad(W, ((0, K_PAD - W.shape[0]), (0, D_PAD - W.shape[1])))


def _pad_b(b):
    return jnp.pad(b, (0, D_PAD - b.shape[0])).reshape(1, D_PAD)


def kernel(edge_sim, edge_rat, x,
           W1s, b1s, W1r, b1r, W1f, b1f,
           W2s, b2s, W2r, b2r, W2f, b2f):
    ss, sd = _prep_edges(edge_sim)
    rs, rd = _prep_edges(edge_rat)
    x_p = jnp.pad(x, ((0, N_PAD - N), (0, 0)))

    hsl, hsh, hrl, hrh, f1 = _mm3(
        (x_p,), (D_IN,), _pad_w(W1s, D_IN), _pad_w(W1r, D_IN),
        _pad_w(W1f, D_IN), _pad_b(b1s), _pad_b(b1r), _pad_b(b1f), D_IN)
    a1l, a1h = _sc_agg(hsl, hsh, hrl, hrh, ss, sd, rs, rd)
    hsl2, hsh2, hrl2, hrh2, f2 = _mm3(
        (f1, a1l, a1h), (D_PAD, HALF, HALF), _pad_w(W2s, D_PAD),
        _pad_w(W2r, D_PAD), _pad_w(W2f, D_PAD), _pad_b(b2s), _pad_b(b2r),
        _pad_b(b2f), D_PAD, relu_combine=True)
    a2l, a2h = _sc_agg(hsl2, hsh2, hrl2, hrh2, ss, sd, rs, rd)
    out = _combine(f2, a2l, a2h)
    return out[:N, :D_HID]
